# initial kernel scaffold (unmeasured)
import jax
import jax.numpy as jnp
from jax import lax
from jax.experimental import pallas as pl
from jax.experimental.pallas import tpu as pltpu

N_DEV = 32
N_TOK = 256
D_IN = 128
D_OUT = 256
N_EXP = 64
EXP_PER = 2
ROWS_PER = N_TOK // N_DEV


def kernel(x, router_W, route_idx, expert_W):
    def body(x_ref, rw_ref, idx_ref, ew_ref, out_ref,
             partial_ref, recv_ref, send_sems, recv_sems):
        my_i = lax.axis_index("i")

        xv = x_ref[:, :]
        scores = jnp.dot(xv, rw_ref[:, :],
                         preferred_element_type=jnp.float32,
                         precision=lax.Precision.HIGHEST)
        s_max = jnp.max(scores, axis=1, keepdims=True)
        p = jnp.exp(scores - s_max)
        probs = p / jnp.sum(p, axis=1, keepdims=True)

        idx0 = idx_ref[:, 0:1]
        idx1 = idx_ref[:, 1:2]
        eiota = lax.broadcasted_iota(jnp.int32, (N_TOK, N_EXP), 1)
        g0 = jnp.sum(jnp.where(eiota == idx0, probs, 0.0), axis=1,
                     keepdims=True)
        g1 = jnp.sum(jnp.where(eiota == idx1, probs, 0.0), axis=1,
                     keepdims=True)
        gs = g0 + g1

        acc = jnp.zeros((N_TOK, D_OUT), jnp.float32)
        for le in range(EXP_PER):
            e_id = EXP_PER * my_i + le
            gate = (jnp.where(idx0 == e_id, g0 / gs, 0.0)
                    + jnp.where(idx1 == e_id, g1 / gs, 0.0))
            y = jnp.dot(xv, ew_ref[le],
                        preferred_element_type=jnp.float32,
                        precision=lax.Precision.HIGHEST)
            acc = acc + gate * y
        partial_ref[:, :] = acc

        recv_ref[my_i, :, :] = lax.dynamic_slice(
            acc, (my_i * ROWS_PER, 0), (ROWS_PER, D_OUT))

        for j in range(N_DEV):
            @pl.when(j != my_i)
            def _():
                rdma = pltpu.make_async_remote_copy(
                    src_ref=partial_ref.at[pl.ds(j * ROWS_PER, ROWS_PER), :],
                    dst_ref=recv_ref.at[my_i],
                    send_sem=send_sems.at[j],
                    recv_sem=recv_sems.at[my_i],
                    device_id=(j,),
                    device_id_type=pl.DeviceIdType.MESH,
                )
                rdma.start()

        for s in range(N_DEV):
            @pl.when(s != my_i)
            def _():
                rdma = pltpu.make_async_remote_copy(
                    src_ref=partial_ref.at[pl.ds(0, ROWS_PER), :],
                    dst_ref=recv_ref.at[s],
                    send_sem=send_sems.at[s],
                    recv_sem=recv_sems.at[s],
                    device_id=(s,),
                    device_id_type=pl.DeviceIdType.MESH,
                )
                rdma.wait_recv()

        out_ref[:, :] = jnp.sum(recv_ref[:, :, :], axis=0)

        for j in range(N_DEV):
            @pl.when(j != my_i)
            def _():
                rdma = pltpu.make_async_remote_copy(
                    src_ref=partial_ref.at[pl.ds(j * ROWS_PER, ROWS_PER), :],
                    dst_ref=recv_ref.at[my_i],
                    send_sem=send_sems.at[j],
                    recv_sem=recv_sems.at[my_i],
                    device_id=(j,),
                    device_id_type=pl.DeviceIdType.MESH,
                )
                rdma.wait_send()

    return pl.pallas_call(
        body,
        out_shape=jax.ShapeDtypeStruct((ROWS_PER, D_OUT), jnp.float32),
        in_specs=[
            pl.BlockSpec(memory_space=pltpu.VMEM),
            pl.BlockSpec(memory_space=pltpu.VMEM),
            pl.BlockSpec(memory_space=pltpu.VMEM),
            pl.BlockSpec(memory_space=pltpu.VMEM),
        ],
        out_specs=pl.BlockSpec(memory_space=pltpu.VMEM),
        scratch_shapes=[
            pltpu.VMEM((N_TOK, D_OUT), jnp.float32),
            pltpu.VMEM((N_DEV, ROWS_PER, D_OUT), jnp.float32),
            pltpu.SemaphoreType.DMA((N_DEV,)),
            pltpu.SemaphoreType.DMA((N_DEV,)),
        ],
    )(x, router_W, route_idx, expert_W)


# baseline (device time: 22265 ns/iter reference)
import jax
import jax.numpy as jnp
from jax import lax
from jax.experimental import pallas as pl
from jax.experimental.pallas import tpu as pltpu

N_DEV = 32
N_TOK = 256
D_IN = 128
D_OUT = 256
N_EXP = 64
EXP_PER = 2
ROWS_PER = N_TOK // N_DEV


def kernel(x, router_W, route_idx, expert_W):
    def body(x_ref, rw_ref, idx_ref, ew_ref, out_ref,
             partial_ref, recv_ref, send_sems, recv_sems):
        my_i = lax.axis_index("i")

        xv = x_ref[:, :]
        scores = jnp.dot(xv, rw_ref[:, :],
                         preferred_element_type=jnp.float32,
                         precision=lax.Precision.HIGHEST)
        s_max = jnp.max(scores, axis=1, keepdims=True)
        p = jnp.exp(scores - s_max)
        probs = p / jnp.sum(p, axis=1, keepdims=True)

        idx0 = idx_ref[:, 0:1]
        idx1 = idx_ref[:, 1:2]
        eiota = lax.broadcasted_iota(jnp.int32, (N_TOK, N_EXP), 1)
        g0 = jnp.sum(jnp.where(eiota == idx0, probs, 0.0), axis=1,
                     keepdims=True)
        g1 = jnp.sum(jnp.where(eiota == idx1, probs, 0.0), axis=1,
                     keepdims=True)
        gs = g0 + g1

        acc = jnp.zeros((N_TOK, D_OUT), jnp.float32)
        for le in range(EXP_PER):
            e_id = EXP_PER * my_i + le
            gate = (jnp.where(idx0 == e_id, g0 / gs, 0.0)
                    + jnp.where(idx1 == e_id, g1 / gs, 0.0))
            y = jnp.dot(xv, ew_ref[le],
                        preferred_element_type=jnp.float32,
                        precision=lax.Precision.HIGHEST)
            acc = acc + gate * y
        partial_ref[:, :] = acc

        recv_ref[my_i, :, :] = partial_ref[pl.ds(my_i * ROWS_PER, ROWS_PER), :]

        for j in range(N_DEV):
            @pl.when(j != my_i)
            def _():
                rdma = pltpu.make_async_remote_copy(
                    src_ref=partial_ref.at[pl.ds(j * ROWS_PER, ROWS_PER), :],
                    dst_ref=recv_ref.at[my_i],
                    send_sem=send_sems.at[j],
                    recv_sem=recv_sems.at[my_i],
                    device_id=(j,),
                    device_id_type=pl.DeviceIdType.MESH,
                )
                rdma.start()

        for s in range(N_DEV):
            @pl.when(s != my_i)
            def _():
                rdma = pltpu.make_async_remote_copy(
                    src_ref=partial_ref.at[pl.ds(0, ROWS_PER), :],
                    dst_ref=recv_ref.at[s],
                    send_sem=send_sems.at[s],
                    recv_sem=recv_sems.at[s],
                    device_id=(s,),
                    device_id_type=pl.DeviceIdType.MESH,
                )
                rdma.wait_recv()

        out_ref[:, :] = jnp.sum(recv_ref[:, :, :], axis=0)

        for j in range(N_DEV):
            @pl.when(j != my_i)
            def _():
                rdma = pltpu.make_async_remote_copy(
                    src_ref=partial_ref.at[pl.ds(j * ROWS_PER, ROWS_PER), :],
                    dst_ref=recv_ref.at[my_i],
                    send_sem=send_sems.at[j],
                    recv_sem=recv_sems.at[my_i],
                    device_id=(j,),
                    device_id_type=pl.DeviceIdType.MESH,
                )
                rdma.wait_send()

    return pl.pallas_call(
        body,
        out_shape=jax.ShapeDtypeStruct((ROWS_PER, D_OUT), jnp.float32),
        in_specs=[
            pl.BlockSpec(memory_space=pltpu.VMEM),
            pl.BlockSpec(memory_space=pltpu.VMEM),
            pl.BlockSpec(memory_space=pltpu.VMEM),
            pl.BlockSpec(memory_space=pltpu.VMEM),
        ],
        out_specs=pl.BlockSpec(memory_space=pltpu.VMEM),
        scratch_shapes=[
            pltpu.VMEM((N_TOK, D_OUT), jnp.float32),
            pltpu.VMEM((N_DEV, ROWS_PER, D_OUT), jnp.float32),
            pltpu.SemaphoreType.DMA((N_DEV,)),
            pltpu.SemaphoreType.DMA((N_DEV,)),
        ],
    )(x, router_W, route_idx, expert_W)


# device time: 16565 ns/iter; 1.3441x vs baseline; 1.3441x over previous
import os

import jax
import jax.numpy as jnp
from jax import lax
from jax.experimental import pallas as pl
from jax.experimental.pallas import tpu as pltpu

VARIANT = os.environ.get("KVAR", "base")
_PREC = (lax.Precision.HIGHEST
         if os.environ.get("KPREC", "highest") == "highest" else None)

N_DEV = 32
N_TOK = 256
D_IN = 128
D_OUT = 256
N_EXP = 64
EXP_PER = 2
ROWS_PER = N_TOK // N_DEV


def kernel(x, router_W, route_idx, expert_W):
    def body(x_ref, rw_ref, idx_ref, ew_ref, out_ref,
             partial_ref, recv_ref, send_sems, recv_sems):
        my_i = lax.axis_index("i")

        xv = x_ref[:, :]
        scores = jnp.dot(xv, rw_ref[:, :],
                         preferred_element_type=jnp.float32,
                         precision=_PREC)
        s_max = jnp.max(scores, axis=1, keepdims=True)
        p = jnp.exp(scores - s_max)
        probs = p / jnp.sum(p, axis=1, keepdims=True)

        idx0 = idx_ref[:, 0:1]
        idx1 = idx_ref[:, 1:2]
        eiota = lax.broadcasted_iota(jnp.int32, (N_TOK, N_EXP), 1)
        g0 = jnp.sum(jnp.where(eiota == idx0, probs, 0.0), axis=1,
                     keepdims=True)
        g1 = jnp.sum(jnp.where(eiota == idx1, probs, 0.0), axis=1,
                     keepdims=True)
        gs = g0 + g1

        acc = jnp.zeros((N_TOK, D_OUT), jnp.float32)
        for le in range(EXP_PER):
            e_id = EXP_PER * my_i + le
            gate = (jnp.where(idx0 == e_id, g0 / gs, 0.0)
                    + jnp.where(idx1 == e_id, g1 / gs, 0.0))
            y = jnp.dot(xv, ew_ref[le],
                        preferred_element_type=jnp.float32,
                        precision=_PREC)
            acc = acc + gate * y
        partial_ref[:, :] = acc

        if VARIANT == "nocomm":
            for s in range(N_DEV):
                recv_ref[s, :, :] = partial_ref[pl.ds(my_i * ROWS_PER,
                                                      ROWS_PER), :]
            out_ref[:, :] = jnp.sum(recv_ref[:, :, :], axis=0)
            return

        if VARIANT == "xbar":
            barrier_sem = pltpu.get_barrier_semaphore()
            for nbr in range(N_DEV):
                @pl.when(nbr != my_i)
                def _():
                    pl.semaphore_signal(
                        barrier_sem, inc=1,
                        device_id=(nbr,),
                        device_id_type=pl.DeviceIdType.MESH,
                    )
            pl.semaphore_wait(barrier_sem, N_DEV - 1)

        recv_ref[my_i, :, :] = partial_ref[pl.ds(my_i * ROWS_PER, ROWS_PER), :]

        for j in range(N_DEV):
            @pl.when(j != my_i)
            def _():
                rdma = pltpu.make_async_remote_copy(
                    src_ref=partial_ref.at[pl.ds(j * ROWS_PER, ROWS_PER), :],
                    dst_ref=recv_ref.at[my_i],
                    send_sem=send_sems.at[j],
                    recv_sem=recv_sems.at[my_i],
                    device_id=(j,),
                    device_id_type=pl.DeviceIdType.MESH,
                )
                rdma.start()

        for s in range(N_DEV):
            @pl.when(s != my_i)
            def _():
                rdma = pltpu.make_async_remote_copy(
                    src_ref=partial_ref.at[pl.ds(0, ROWS_PER), :],
                    dst_ref=recv_ref.at[s],
                    send_sem=send_sems.at[s],
                    recv_sem=recv_sems.at[s],
                    device_id=(s,),
                    device_id_type=pl.DeviceIdType.MESH,
                )
                rdma.wait_recv()

        out_ref[:, :] = jnp.sum(recv_ref[:, :, :], axis=0)

        for j in range(N_DEV):
            @pl.when(j != my_i)
            def _():
                rdma = pltpu.make_async_remote_copy(
                    src_ref=partial_ref.at[pl.ds(j * ROWS_PER, ROWS_PER), :],
                    dst_ref=recv_ref.at[my_i],
                    send_sem=send_sems.at[j],
                    recv_sem=recv_sems.at[my_i],
                    device_id=(j,),
                    device_id_type=pl.DeviceIdType.MESH,
                )
                rdma.wait_send()

    params = {}
    if VARIANT == "xbar":
        CP = getattr(pltpu, "CompilerParams", None) or pltpu.TPUCompilerParams
        params["compiler_params"] = CP(collective_id=0)
    return pl.pallas_call(
        body,
        out_shape=jax.ShapeDtypeStruct((ROWS_PER, D_OUT), jnp.float32),
        **params,
        in_specs=[
            pl.BlockSpec(memory_space=pltpu.VMEM),
            pl.BlockSpec(memory_space=pltpu.VMEM),
            pl.BlockSpec(memory_space=pltpu.VMEM),
            pl.BlockSpec(memory_space=pltpu.VMEM),
        ],
        out_specs=pl.BlockSpec(memory_space=pltpu.VMEM),
        scratch_shapes=[
            pltpu.VMEM((N_TOK, D_OUT), jnp.float32),
            pltpu.VMEM((N_DEV, ROWS_PER, D_OUT), jnp.float32),
            pltpu.SemaphoreType.DMA((N_DEV,)),
            pltpu.SemaphoreType.DMA((N_DEV,)),
        ],
    )(x, router_W, route_idx, expert_W)


# device time: 14513 ns/iter; 1.5341x vs baseline; 1.1414x over previous
import os

import jax
import jax.numpy as jnp
from jax import lax
from jax.experimental import pallas as pl
from jax.experimental.pallas import tpu as pltpu

VARIANT = os.environ.get("KVAR", "base")
_PREC = (lax.Precision.HIGHEST
         if os.environ.get("KPREC", "highest") == "highest" else None)

N_DEV = 32
N_TOK = 256
D_IN = 128
D_OUT = 256
N_EXP = 64
EXP_PER = 2
ROWS_PER = N_TOK // N_DEV


def kernel(x, router_W, route_idx, expert_W):
    use_xbar = VARIANT in ("xbar", "sparse")

    def body(x_ref, rw_ref, idx_ref, ew_ref, out_ref,
             partial_ref, recv_ref, send_sems, recv_sems):
        my_i = lax.axis_index("i")

        if use_xbar:
            barrier_sem = pltpu.get_barrier_semaphore()
            for nbr in range(N_DEV):
                @pl.when(nbr != my_i)
                def _():
                    pl.semaphore_signal(
                        barrier_sem, inc=1,
                        device_id=(nbr,),
                        device_id_type=pl.DeviceIdType.MESH,
                    )

        xv = x_ref[:, :]
        scores = jnp.dot(xv, rw_ref[:, :],
                         preferred_element_type=jnp.float32,
                         precision=_PREC)
        s_max = jnp.max(scores, axis=1, keepdims=True)
        p = jnp.exp(scores - s_max)
        probs = p / jnp.sum(p, axis=1, keepdims=True)

        idx0 = idx_ref[:, 0:1]
        idx1 = idx_ref[:, 1:2]
        eiota = lax.broadcasted_iota(jnp.int32, (N_TOK, N_EXP), 1)
        g0 = jnp.sum(jnp.where(eiota == idx0, probs, 0.0), axis=1,
                     keepdims=True)
        g1 = jnp.sum(jnp.where(eiota == idx1, probs, 0.0), axis=1,
                     keepdims=True)
        gs = g0 + g1

        acc = jnp.zeros((N_TOK, D_OUT), jnp.float32)
        for le in range(EXP_PER):
            e_id = EXP_PER * my_i + le
            gate = (jnp.where(idx0 == e_id, g0 / gs, 0.0)
                    + jnp.where(idx1 == e_id, g1 / gs, 0.0))
            y = jnp.dot(xv, ew_ref[le],
                        preferred_element_type=jnp.float32,
                        precision=_PREC)
            acc = acc + gate * y
        partial_ref[:, :] = acc

        if VARIANT == "nocomm":
            for s in range(N_DEV):
                recv_ref[s, :, :] = partial_ref[pl.ds(my_i * ROWS_PER,
                                                      ROWS_PER), :]
            out_ref[:, :] = jnp.sum(recv_ref[:, :, :], axis=0)
            return

        if use_xbar:
            pl.semaphore_wait(barrier_sem, N_DEV - 1)

        recv_ref[my_i, :, :] = partial_ref[pl.ds(my_i * ROWS_PER, ROWS_PER), :]

        if VARIANT == "sparse":
            chip0 = idx_ref[:, 0:1] // EXP_PER
            chip1 = idx_ref[:, 1:2] // EXP_PER
            m = ((chip0 == my_i) | (chip1 == my_i)).astype(jnp.int32)
            msum = jnp.sum(m.reshape(N_DEV, ROWS_PER), axis=1,
                           keepdims=True)
            mine = idx_ref[pl.ds(my_i * ROWS_PER, ROWS_PER), :] // EXP_PER
            siota = lax.broadcasted_iota(jnp.int32, (ROWS_PER, N_DEV), 1)
            r = ((mine[:, 0:1] == siota) | (mine[:, 1:2] == siota))
            rsum = jnp.sum(r.astype(jnp.int32), axis=0,
                           keepdims=True)
            send_pred = [msum[j, 0] > 0 for j in range(N_DEV)]
            recv_pred = [rsum[0, s] > 0 for s in range(N_DEV)]
        else:
            send_pred = [True] * N_DEV
            recv_pred = [True] * N_DEV

        for j in range(N_DEV):
            @pl.when((j != my_i) & send_pred[j])
            def _():
                rdma = pltpu.make_async_remote_copy(
                    src_ref=partial_ref.at[pl.ds(j * ROWS_PER, ROWS_PER), :],
                    dst_ref=recv_ref.at[my_i],
                    send_sem=send_sems.at[j],
                    recv_sem=recv_sems.at[my_i],
                    device_id=(j,),
                    device_id_type=pl.DeviceIdType.MESH,
                )
                rdma.start()

        for s in range(N_DEV):
            @pl.when((s != my_i) & recv_pred[s])
            def _():
                rdma = pltpu.make_async_remote_copy(
                    src_ref=partial_ref.at[pl.ds(0, ROWS_PER), :],
                    dst_ref=recv_ref.at[s],
                    send_sem=send_sems.at[s],
                    recv_sem=recv_sems.at[s],
                    device_id=(s,),
                    device_id_type=pl.DeviceIdType.MESH,
                )
                rdma.wait_recv()
            if VARIANT == "sparse":
                @pl.when((s != my_i) & jnp.logical_not(recv_pred[s]))
                def _():
                    recv_ref[s, :, :] = jnp.zeros((ROWS_PER, D_OUT),
                                                  jnp.float32)

        out_ref[:, :] = jnp.sum(recv_ref[:, :, :], axis=0)

        for j in range(N_DEV):
            @pl.when((j != my_i) & send_pred[j])
            def _():
                rdma = pltpu.make_async_remote_copy(
                    src_ref=partial_ref.at[pl.ds(j * ROWS_PER, ROWS_PER), :],
                    dst_ref=recv_ref.at[my_i],
                    send_sem=send_sems.at[j],
                    recv_sem=recv_sems.at[my_i],
                    device_id=(j,),
                    device_id_type=pl.DeviceIdType.MESH,
                )
                rdma.wait_send()

    params = {}
    if use_xbar:
        CP = getattr(pltpu, "CompilerParams", None) or pltpu.TPUCompilerParams
        params["compiler_params"] = CP(collective_id=0)
    return pl.pallas_call(
        body,
        out_shape=jax.ShapeDtypeStruct((ROWS_PER, D_OUT), jnp.float32),
        **params,
        in_specs=[
            pl.BlockSpec(memory_space=pltpu.VMEM),
            pl.BlockSpec(memory_space=pltpu.VMEM),
            pl.BlockSpec(memory_space=pltpu.VMEM),
            pl.BlockSpec(memory_space=pltpu.VMEM),
        ],
        out_specs=pl.BlockSpec(memory_space=pltpu.VMEM),
        scratch_shapes=[
            pltpu.VMEM((N_TOK, D_OUT), jnp.float32),
            pltpu.VMEM((N_DEV, ROWS_PER, D_OUT), jnp.float32),
            pltpu.SemaphoreType.DMA((N_DEV,)),
            pltpu.SemaphoreType.DMA((N_DEV,)),
        ],
    )(x, router_W, route_idx, expert_W)


# device time: 14448 ns/iter; 1.5410x vs baseline; 1.0045x over previous
import os

import jax
import jax.numpy as jnp
from jax import lax
from jax.experimental import pallas as pl
from jax.experimental.pallas import tpu as pltpu

VARIANT = os.environ.get("KVAR", "base")
_PREC = (lax.Precision.HIGHEST
         if os.environ.get("KPREC", "highest") == "highest" else None)

N_DEV = 32
N_TOK = 256
D_IN = 128
D_OUT = 256
N_EXP = 64
EXP_PER = 2
ROWS_PER = N_TOK // N_DEV


def kernel(x, router_W, route_idx, expert_W):
    use_xbar = VARIANT in ("xbar", "sparse", "sparse2")

    def body(x_ref, rw_ref, idx_ref, ew_ref, out_ref,
             partial_ref, recv_ref, send_sems, recv_sems):
        my_i = lax.axis_index("i")

        if use_xbar:
            barrier_sem = pltpu.get_barrier_semaphore()
            for nbr in range(N_DEV):
                @pl.when(nbr != my_i)
                def _():
                    pl.semaphore_signal(
                        barrier_sem, inc=1,
                        device_id=(nbr,),
                        device_id_type=pl.DeviceIdType.MESH,
                    )

        xv = x_ref[:, :]
        scores = jnp.dot(xv, rw_ref[:, :],
                         preferred_element_type=jnp.float32,
                         precision=_PREC)
        s_max = jnp.max(scores, axis=1, keepdims=True)
        p = jnp.exp(scores - s_max)
        probs = p / jnp.sum(p, axis=1, keepdims=True)

        idx0 = idx_ref[:, 0:1]
        idx1 = idx_ref[:, 1:2]
        eiota = lax.broadcasted_iota(jnp.int32, (N_TOK, N_EXP), 1)
        g0 = jnp.sum(jnp.where(eiota == idx0, probs, 0.0), axis=1,
                     keepdims=True)
        g1 = jnp.sum(jnp.where(eiota == idx1, probs, 0.0), axis=1,
                     keepdims=True)
        gs = g0 + g1

        acc = jnp.zeros((N_TOK, D_OUT), jnp.float32)
        for le in range(EXP_PER):
            e_id = EXP_PER * my_i + le
            gate = (jnp.where(idx0 == e_id, g0 / gs, 0.0)
                    + jnp.where(idx1 == e_id, g1 / gs, 0.0))
            y = jnp.dot(xv, ew_ref[le],
                        preferred_element_type=jnp.float32,
                        precision=_PREC)
            acc = acc + gate * y
        partial_ref[:, :] = acc

        if VARIANT == "nocomm":
            for s in range(N_DEV):
                recv_ref[s, :, :] = partial_ref[pl.ds(my_i * ROWS_PER,
                                                      ROWS_PER), :]
            out_ref[:, :] = jnp.sum(recv_ref[:, :, :], axis=0)
            return

        if VARIANT in ("sparse", "sparse2"):
            chip0 = idx_ref[:, 0:1] // EXP_PER
            chip1 = idx_ref[:, 1:2] // EXP_PER
            m = ((chip0 == my_i) | (chip1 == my_i)).astype(jnp.int32)
            msum = jnp.sum(m.reshape(N_DEV, ROWS_PER), axis=1,
                           keepdims=True)
            mine = idx_ref[pl.ds(my_i * ROWS_PER, ROWS_PER), :] // EXP_PER
            siota = lax.broadcasted_iota(jnp.int32, (ROWS_PER, N_DEV), 1)
            r = ((mine[:, 0:1] == siota) | (mine[:, 1:2] == siota))
            rsum = jnp.sum(r.astype(jnp.int32), axis=0,
                           keepdims=True)
            send_pred = [msum[j, 0] > 0 for j in range(N_DEV)]
            recv_pred = [rsum[0, s] > 0 for s in range(N_DEV)]
        else:
            send_pred = [True] * N_DEV
            recv_pred = [True] * N_DEV

        if VARIANT != "sparse2":
            recv_ref[my_i, :, :] = partial_ref[pl.ds(my_i * ROWS_PER,
                                                     ROWS_PER), :]

        if use_xbar:
            pl.semaphore_wait(barrier_sem, N_DEV - 1)

        for j in range(N_DEV):
            @pl.when((j != my_i) & send_pred[j])
            def _():
                rdma = pltpu.make_async_remote_copy(
                    src_ref=partial_ref.at[pl.ds(j * ROWS_PER, ROWS_PER), :],
                    dst_ref=recv_ref.at[my_i],
                    send_sem=send_sems.at[j],
                    recv_sem=recv_sems.at[my_i],
                    device_id=(j,),
                    device_id_type=pl.DeviceIdType.MESH,
                )
                rdma.start()

        if VARIANT == "sparse2":
            out_val = partial_ref[pl.ds(my_i * ROWS_PER, ROWS_PER), :]
            zeros = jnp.zeros((ROWS_PER, D_OUT), jnp.float32)
            for s in range(N_DEV):
                take = (s != my_i) & recv_pred[s]

                @pl.when(take)
                def _():
                    rdma = pltpu.make_async_remote_copy(
                        src_ref=partial_ref.at[pl.ds(0, ROWS_PER), :],
                        dst_ref=recv_ref.at[s],
                        send_sem=send_sems.at[s],
                        recv_sem=recv_sems.at[s],
                        device_id=(s,),
                        device_id_type=pl.DeviceIdType.MESH,
                    )
                    rdma.wait_recv()
                out_val = out_val + jnp.where(take, recv_ref[s, :, :], zeros)
            out_ref[:, :] = out_val
        else:
            for s in range(N_DEV):
                @pl.when((s != my_i) & recv_pred[s])
                def _():
                    rdma = pltpu.make_async_remote_copy(
                        src_ref=partial_ref.at[pl.ds(0, ROWS_PER), :],
                        dst_ref=recv_ref.at[s],
                        send_sem=send_sems.at[s],
                        recv_sem=recv_sems.at[s],
                        device_id=(s,),
                        device_id_type=pl.DeviceIdType.MESH,
                    )
                    rdma.wait_recv()
                if VARIANT == "sparse":
                    @pl.when((s != my_i) & jnp.logical_not(recv_pred[s]))
                    def _():
                        recv_ref[s, :, :] = jnp.zeros((ROWS_PER, D_OUT),
                                                      jnp.float32)

            out_ref[:, :] = jnp.sum(recv_ref[:, :, :], axis=0)

        for j in range(N_DEV):
            @pl.when((j != my_i) & send_pred[j])
            def _():
                rdma = pltpu.make_async_remote_copy(
                    src_ref=partial_ref.at[pl.ds(j * ROWS_PER, ROWS_PER), :],
                    dst_ref=recv_ref.at[my_i],
                    send_sem=send_sems.at[j],
                    recv_sem=recv_sems.at[my_i],
                    device_id=(j,),
                    device_id_type=pl.DeviceIdType.MESH,
                )
                rdma.wait_send()

    params = {}
    if use_xbar:
        CP = getattr(pltpu, "CompilerParams", None) or pltpu.TPUCompilerParams
        params["compiler_params"] = CP(collective_id=0)
    return pl.pallas_call(
        body,
        out_shape=jax.ShapeDtypeStruct((ROWS_PER, D_OUT), jnp.float32),
        **params,
        in_specs=[
            pl.BlockSpec(memory_space=pltpu.VMEM),
            pl.BlockSpec(memory_space=pltpu.VMEM),
            pl.BlockSpec(memory_space=pltpu.VMEM),
            pl.BlockSpec(memory_space=pltpu.VMEM),
        ],
        out_specs=pl.BlockSpec(memory_space=pltpu.VMEM),
        scratch_shapes=[
            pltpu.VMEM((N_TOK, D_OUT), jnp.float32),
            pltpu.VMEM((N_DEV, ROWS_PER, D_OUT), jnp.float32),
            pltpu.SemaphoreType.DMA((N_DEV,)),
            pltpu.SemaphoreType.DMA((N_DEV,)),
        ],
    )(x, router_W, route_idx, expert_W)


# device time: 14315 ns/iter; 1.5554x vs baseline; 1.0093x over previous
import os

import jax
import jax.numpy as jnp
from jax import lax
from jax.experimental import pallas as pl
from jax.experimental.pallas import tpu as pltpu

VARIANT = os.environ.get("KVAR", "sparse2")
_PREC = (lax.Precision.HIGHEST
         if os.environ.get("KPREC", "default") == "highest" else None)

N_DEV = 32
N_TOK = 256
D_IN = 128
D_OUT = 256
N_EXP = 64
EXP_PER = 2
ROWS_PER = N_TOK // N_DEV


def kernel(x, router_W, route_idx, expert_W):
    use_xbar = VARIANT in ("xbar", "sparse", "sparse2")

    def body(x_ref, rw_ref, idx_ref, ew_ref, out_ref,
             partial_ref, recv_ref, send_sems, recv_sems):
        my_i = lax.axis_index("i")

        if use_xbar:
            barrier_sem = pltpu.get_barrier_semaphore()
            for nbr in range(N_DEV):
                @pl.when(nbr != my_i)
                def _():
                    pl.semaphore_signal(
                        barrier_sem, inc=1,
                        device_id=(nbr,),
                        device_id_type=pl.DeviceIdType.MESH,
                    )

        xv = x_ref[:, :]
        scores = jnp.dot(xv, rw_ref[:, :],
                         preferred_element_type=jnp.float32,
                         precision=_PREC)
        s_max = jnp.max(scores, axis=1, keepdims=True)
        p = jnp.exp(scores - s_max)
        probs = p / jnp.sum(p, axis=1, keepdims=True)

        idx0 = idx_ref[:, 0:1]
        idx1 = idx_ref[:, 1:2]
        eiota = lax.broadcasted_iota(jnp.int32, (N_TOK, N_EXP), 1)
        g0 = jnp.sum(jnp.where(eiota == idx0, probs, 0.0), axis=1,
                     keepdims=True)
        g1 = jnp.sum(jnp.where(eiota == idx1, probs, 0.0), axis=1,
                     keepdims=True)
        gs = g0 + g1

        acc = jnp.zeros((N_TOK, D_OUT), jnp.float32)
        for le in range(EXP_PER):
            e_id = EXP_PER * my_i + le
            gate = (jnp.where(idx0 == e_id, g0 / gs, 0.0)
                    + jnp.where(idx1 == e_id, g1 / gs, 0.0))
            y = jnp.dot(xv, ew_ref[le],
                        preferred_element_type=jnp.float32,
                        precision=_PREC)
            acc = acc + gate * y
        partial_ref[:, :] = acc

        if VARIANT == "nocomm":
            for s in range(N_DEV):
                recv_ref[s, :, :] = partial_ref[pl.ds(my_i * ROWS_PER,
                                                      ROWS_PER), :]
            out_ref[:, :] = jnp.sum(recv_ref[:, :, :], axis=0)
            return

        if VARIANT in ("sparse", "sparse2"):
            chip0 = idx_ref[:, 0:1] // EXP_PER
            chip1 = idx_ref[:, 1:2] // EXP_PER
            m = ((chip0 == my_i) | (chip1 == my_i)).astype(jnp.int32)
            msum = jnp.sum(m.reshape(N_DEV, ROWS_PER), axis=1,
                           keepdims=True)
            mine = idx_ref[pl.ds(my_i * ROWS_PER, ROWS_PER), :] // EXP_PER
            siota = lax.broadcasted_iota(jnp.int32, (ROWS_PER, N_DEV), 1)
            r = ((mine[:, 0:1] == siota) | (mine[:, 1:2] == siota))
            rsum = jnp.sum(r.astype(jnp.int32), axis=0,
                           keepdims=True)
            send_pred = [msum[j, 0] > 0 for j in range(N_DEV)]
            recv_pred = [rsum[0, s] > 0 for s in range(N_DEV)]
        else:
            send_pred = [True] * N_DEV
            recv_pred = [True] * N_DEV

        if VARIANT != "sparse2":
            recv_ref[my_i, :, :] = partial_ref[pl.ds(my_i * ROWS_PER,
                                                     ROWS_PER), :]

        if use_xbar:
            pl.semaphore_wait(barrier_sem, N_DEV - 1)

        for j in range(N_DEV):
            @pl.when((j != my_i) & send_pred[j])
            def _():
                rdma = pltpu.make_async_remote_copy(
                    src_ref=partial_ref.at[pl.ds(j * ROWS_PER, ROWS_PER), :],
                    dst_ref=recv_ref.at[my_i],
                    send_sem=send_sems.at[j],
                    recv_sem=recv_sems.at[my_i],
                    device_id=(j,),
                    device_id_type=pl.DeviceIdType.MESH,
                )
                rdma.start()

        if VARIANT == "sparse2":
            out_val = partial_ref[pl.ds(my_i * ROWS_PER, ROWS_PER), :]
            zeros = jnp.zeros((ROWS_PER, D_OUT), jnp.float32)
            for s in range(N_DEV):
                take = (s != my_i) & recv_pred[s]

                @pl.when(take)
                def _():
                    rdma = pltpu.make_async_remote_copy(
                        src_ref=partial_ref.at[pl.ds(0, ROWS_PER), :],
                        dst_ref=recv_ref.at[s],
                        send_sem=send_sems.at[s],
                        recv_sem=recv_sems.at[s],
                        device_id=(s,),
                        device_id_type=pl.DeviceIdType.MESH,
                    )
                    rdma.wait_recv()
                out_val = out_val + jnp.where(take, recv_ref[s, :, :], zeros)
            out_ref[:, :] = out_val
        else:
            for s in range(N_DEV):
                @pl.when((s != my_i) & recv_pred[s])
                def _():
                    rdma = pltpu.make_async_remote_copy(
                        src_ref=partial_ref.at[pl.ds(0, ROWS_PER), :],
                        dst_ref=recv_ref.at[s],
                        send_sem=send_sems.at[s],
                        recv_sem=recv_sems.at[s],
                        device_id=(s,),
                        device_id_type=pl.DeviceIdType.MESH,
                    )
                    rdma.wait_recv()
                if VARIANT == "sparse":
                    @pl.when((s != my_i) & jnp.logical_not(recv_pred[s]))
                    def _():
                        recv_ref[s, :, :] = jnp.zeros((ROWS_PER, D_OUT),
                                                      jnp.float32)

            out_ref[:, :] = jnp.sum(recv_ref[:, :, :], axis=0)

        for j in range(N_DEV):
            @pl.when((j != my_i) & send_pred[j])
            def _():
                rdma = pltpu.make_async_remote_copy(
                    src_ref=partial_ref.at[pl.ds(j * ROWS_PER, ROWS_PER), :],
                    dst_ref=recv_ref.at[my_i],
                    send_sem=send_sems.at[j],
                    recv_sem=recv_sems.at[my_i],
                    device_id=(j,),
                    device_id_type=pl.DeviceIdType.MESH,
                )
                rdma.wait_send()

    params = {}
    if use_xbar:
        CP = getattr(pltpu, "CompilerParams", None) or pltpu.TPUCompilerParams
        params["compiler_params"] = CP(collective_id=0)
    return pl.pallas_call(
        body,
        out_shape=jax.ShapeDtypeStruct((ROWS_PER, D_OUT), jnp.float32),
        **params,
        in_specs=[
            pl.BlockSpec(memory_space=pltpu.VMEM),
            pl.BlockSpec(memory_space=pltpu.VMEM),
            pl.BlockSpec(memory_space=pltpu.VMEM),
            pl.BlockSpec(memory_space=pltpu.VMEM),
        ],
        out_specs=pl.BlockSpec(memory_space=pltpu.VMEM),
        scratch_shapes=[
            pltpu.VMEM((N_TOK, D_OUT), jnp.float32),
            pltpu.VMEM((N_DEV, ROWS_PER, D_OUT), jnp.float32),
            pltpu.SemaphoreType.DMA((N_DEV,)),
            pltpu.SemaphoreType.DMA((N_DEV,)),
        ],
    )(x, router_W, route_idx, expert_W)
